# baseline (speedup 1.0000x reference)
import functools

import jax
import jax.numpy as jnp
from jax import lax
from jax.experimental import pallas as pl
from jax.experimental.pallas import tpu as pltpu
from jax.experimental.pallas import tpu_sc as plsc

BATCH = 16384
D_INST = 64
D_LIGHT = 32
D_APP = 32

_NC = 2
_NS = 16
NW = _NC * _NS
BPW = BATCH // NW
L = 16

_MESH = plsc.VectorSubcoreMesh(core_axis_name="c", subcore_axis_name="s")


def _gather_one_table(w_hbm, idx_hbm, out_hbm, idx_v, staging, sem, base):
    del staging
    pltpu.sync_copy(idx_hbm.at[pl.ds(base, BPW)], idx_v)

    def issue_body(jb, _):
        rvec = idx_v[pl.ds(jb * L, L)]
        for j2 in range(L):
            pltpu.async_copy(
                w_hbm.at[rvec[j2]], out_hbm.at[base + jb * L + j2], sem
            )
        return 0

    lax.fori_loop(0, BPW // L, issue_body, 0)

    def drain_body(jb, _):
        rvec = idx_v[pl.ds(jb * L, L)]
        for j2 in range(L):
            pltpu.make_async_copy(
                w_hbm.at[rvec[j2]], out_hbm.at[base + jb * L + j2], sem
            ).wait()
        return 0

    lax.fori_loop(0, BPW // L, drain_body, 0)


@functools.partial(
    pl.kernel,
    mesh=_MESH,
    out_type=(
        jax.ShapeDtypeStruct((BATCH, D_INST), jnp.float32),
        jax.ShapeDtypeStruct((BATCH, D_LIGHT), jnp.float32),
        jax.ShapeDtypeStruct((BATCH, D_APP), jnp.float32),
    ),
    scratch_types=[
        pltpu.VMEM((BPW,), jnp.int32),
        pltpu.VMEM((BPW, D_INST), jnp.float32),
        pltpu.VMEM((BPW, D_LIGHT), jnp.float32),
        pltpu.SemaphoreType.DMA,
    ],
)
def _gather3(inst_hbm, light_hbm, frame_hbm, wi_hbm, wl_hbm, wa_hbm,
             out_i, out_l, out_a,
             idx_v, stage_i, stage_s, sem):
    wid = lax.axis_index("s") * _NC + lax.axis_index("c")
    base = wid * BPW
    _gather_one_table(wi_hbm, inst_hbm, out_i, idx_v, stage_i, sem, base)
    _gather_one_table(wl_hbm, light_hbm, out_l, idx_v, stage_s, sem, base)
    _gather_one_table(wa_hbm, frame_hbm, out_a, idx_v, stage_s, sem, base)


def kernel(instance_ids, light_env_ids, frame_ids, W_inst, W_light, W_app):
    inst = jnp.squeeze(instance_ids).astype(jnp.int32)
    light = jnp.squeeze(light_env_ids).astype(jnp.int32)
    frame = jnp.squeeze(frame_ids).astype(jnp.int32)
    return _gather3(inst, light, frame, W_inst, W_light, W_app)


# speedup vs baseline: 1.6700x; 1.6700x over previous
import functools

import jax
import jax.numpy as jnp
from jax import lax
from jax.experimental import pallas as pl
from jax.experimental.pallas import tpu as pltpu
from jax.experimental.pallas import tpu_sc as plsc

BATCH = 16384

_NC = 2
_NS = 16
NW = _NC * _NS
BPW = BATCH // NW
CHUNK = 128
NCH = BPW // CHUNK

_MESH = plsc.VectorSubcoreMesh(core_axis_name="c", subcore_axis_name="s")


def _make_gather(d):
    @functools.partial(
        pl.kernel,
        mesh=_MESH,
        compiler_params=pltpu.CompilerParams(use_tc_tiling_on_sc=False),
        out_type=jax.ShapeDtypeStruct((BATCH, d), jnp.float32),
        scratch_types=[
            pltpu.VMEM((BPW,), jnp.int32),
            pltpu.VMEM((BPW, d), jnp.float32),
            pltpu.SemaphoreType.DMA,
        ],
    )
    def gather_one(idx_hbm, w_hbm, out_hbm, idx_v, rows_v, sem):
        wid = lax.axis_index("s") * _NC + lax.axis_index("c")
        base = wid * BPW
        pltpu.sync_copy(idx_hbm.at[pl.ds(base, BPW)], idx_v)
        copies = []
        for c in range(NCH):
            sl = pl.ds(c * CHUNK, CHUNK)
            copies.append(
                pltpu.async_copy(w_hbm.at[idx_v.at[sl]], rows_v.at[sl], sem))
        for cp in copies:
            cp.wait()
        pltpu.sync_copy(rows_v, out_hbm.at[pl.ds(base, BPW)])

    return gather_one


_gather_64 = _make_gather(64)
_gather_32 = _make_gather(32)


def kernel(instance_ids, light_env_ids, frame_ids, W_inst, W_light, W_app):
    inst = jnp.squeeze(instance_ids).astype(jnp.int32)
    light = jnp.squeeze(light_env_ids).astype(jnp.int32)
    frame = jnp.squeeze(frame_ids).astype(jnp.int32)
    out_i = _gather_64(inst, W_inst)
    out_l = _gather_32(light, W_light)
    out_a = _gather_32(frame, W_app)
    return (out_i, out_l, out_a)
